# baseline (speedup 1.0000x reference)
import jax
import jax.numpy as jnp
from jax.experimental import pallas as pl
from jax.experimental.pallas import tpu as pltpu

_D_MODEL = 512
_D_INNER = 1024
_D_STATE = 16
_DT_RANK = 32
_D_CONV = 4
_T = 128


def _silu(v):
    return v * (1.0 / (1.0 + jnp.exp(-v)))


def _softplus(v):
    return jnp.maximum(v, 0.0) + jnp.log1p(jnp.exp(-jnp.abs(v)))


def _mamba_kernel(x_ref, g_ref, b_ref, inTz_ref, wck_ref, cb_ref, xpT_ref,
                  dtT_ref, dtb_ref, An_ref, Dp_ref, outT_ref, msel_ref,
                  p0_ref, dsel_ref, o_ref, carry_ref, h_ref, dA_ref, H_ref):
    lc = pl.program_id(1)

    @pl.when(lc == 0)
    def _():
        carry_ref[...] = jnp.zeros_like(carry_ref)
        h_ref[...] = jnp.zeros_like(h_ref)

    xb = x_ref[0]

    mu = jnp.mean(xb, axis=-1, keepdims=True)
    xc = xb - mu
    var = jnp.mean(xc * xc, axis=-1, keepdims=True)
    xn = xc * jax.lax.rsqrt(var + 1e-5) * g_ref[...] + b_ref[...]

    zg = jnp.dot(xn, inTz_ref[...], preferred_element_type=jnp.float32)

    x_ext = jnp.concatenate([carry_ref[5:8], xn],
                            axis=0).astype(jnp.bfloat16)
    carry_ref[5:8] = xn[_T - 3:, :]
    acc = jnp.broadcast_to(cb_ref[...], (_T, _D_INNER))
    for k in range(_D_CONV):
        acc = acc + jnp.dot(x_ext[k:k + _T], wck_ref[k],
                            preferred_element_type=jnp.float32)
    xcv = _silu(acc)

    x_dbl = jnp.dot(xcv, xpT_ref[...], preferred_element_type=jnp.float32)
    dtr = x_dbl[:, :_DT_RANK]
    Bc = x_dbl[:, _DT_RANK:_DT_RANK + _D_STATE]
    Cc = x_dbl[:, _DT_RANK + _D_STATE:2 * _D_STATE + _DT_RANK]

    dt_lin = jnp.dot(dtr, dtT_ref[...],
                     preferred_element_type=jnp.float32) + dtb_ref[...]
    dt_c = _softplus(dt_lin)
    dtx = dt_c * xcv

    _SC = 32
    An_v = An_ref[...]
    for t0 in range(0, _T, _SC):
        sl = slice(t0, t0 + _SC)
        dA_ref[sl, :_D_STATE] = jnp.exp(dt_c[sl, None, :] * An_v[None, :, :])

    q = jnp.dot(p0_ref[...], Bc, preferred_element_type=jnp.float32)
    z2 = q * dsel_ref[...]
    bg = jnp.dot(z2, jnp.ones((_D_STATE, _T), jnp.float32),
                 preferred_element_type=jnp.float32)
    mb = bg * p0_ref[...]
    dbx = jnp.dot(mb, dtx, preferred_element_type=jnp.float32)
    dA_ref[:, _D_STATE:, :] = dbx.reshape(_T, _D_STATE, _D_INNER)

    def sbody(t, h):
        ab = dA_ref[t]
        h2 = h * ab[:_D_STATE] + ab[_D_STATE:]
        H_ref[t] = h2
        return h2

    h_fin = jax.lax.fori_loop(0, _T, sbody, h_ref[...], unroll=16)
    h_ref[...] = h_fin

    cg8 = jnp.concatenate([Cc] * 8, axis=1)
    cgather = jnp.tile(cg8, (1, _D_STATE))
    mm = msel_ref[...] * cgather
    h_flat = H_ref[...].reshape(_T * _D_STATE, _D_INNER)
    ys = jnp.dot(mm, h_flat, preferred_element_type=jnp.float32)
    yy = ys + xcv * Dp_ref[...]
    yg = yy * _silu(zg)
    o_ref[0] = jnp.dot(yg, outT_ref[...], preferred_element_type=jnp.float32)


def kernel(x, ln_gamma, ln_beta, in_proj_w, conv_w, conv_b, x_proj_w,
           dt_proj_w, dt_proj_b, A_log, D_param, out_proj_w):
    B, L, D = x.shape
    n_chunks = L // _T

    inTz = in_proj_w[_D_INNER:, :].T
    Wx = in_proj_w[:_D_INNER, :].T
    wck = jnp.stack([Wx * conv_w[None, :, 0, k] for k in range(_D_CONV)],
                    axis=0).astype(jnp.bfloat16)
    xpT = x_proj_w.T
    dtT = dt_proj_w.T
    outT = out_proj_w.T
    An = -jnp.exp(A_log).T
    g2 = ln_gamma.reshape(1, D)
    b2 = ln_beta.reshape(1, D)
    cb2 = conv_b.reshape(1, _D_INNER)
    dtb2 = dt_proj_b.reshape(1, _D_INNER)
    Dp2 = D_param.reshape(1, _D_INNER)
    rows = jnp.arange(_T * _D_STATE)
    msel = (rows[None, :] // _D_STATE
            == jnp.arange(_T)[:, None]).astype(jnp.float32)
    p0 = (rows[:, None] // _D_STATE
          == jnp.arange(_T)[None, :]).astype(jnp.float32)
    dsel = (rows[:, None] % _D_STATE
            == jnp.arange(_D_STATE)[None, :]).astype(jnp.float32)

    full = lambda b, l: (0, 0)
    full3 = lambda b, l: (0, 0, 0)
    out = pl.pallas_call(
        _mamba_kernel,
        out_shape=jax.ShapeDtypeStruct((B, L, D), jnp.float32),
        grid=(B, n_chunks),
        in_specs=[
            pl.BlockSpec((1, _T, D), lambda b, l: (b, l, 0)),
            pl.BlockSpec((1, D), full),
            pl.BlockSpec((1, D), full),
            pl.BlockSpec((D, _D_INNER), full),
            pl.BlockSpec((_D_CONV, D, _D_INNER), full3),
            pl.BlockSpec((1, _D_INNER), full),
            pl.BlockSpec((_D_INNER, _DT_RANK + 2 * _D_STATE), full),
            pl.BlockSpec((_DT_RANK, _D_INNER), full),
            pl.BlockSpec((1, _D_INNER), full),
            pl.BlockSpec((_D_STATE, _D_INNER), full),
            pl.BlockSpec((1, _D_INNER), full),
            pl.BlockSpec((_D_INNER, D), full),
            pl.BlockSpec((_T, _T * _D_STATE), full),
            pl.BlockSpec((_T * _D_STATE, _T), full),
            pl.BlockSpec((_T * _D_STATE, _D_STATE), full),
        ],
        out_specs=pl.BlockSpec((1, _T, D), lambda b, l: (b, l, 0)),
        scratch_shapes=[
            pltpu.VMEM((8, _D_MODEL), jnp.float32),
            pltpu.VMEM((_D_STATE, _D_INNER), jnp.float32),
            pltpu.VMEM((_T, 2 * _D_STATE, _D_INNER), jnp.float32),
            pltpu.VMEM((_T, _D_STATE, _D_INNER), jnp.float32),
        ],
        compiler_params=pltpu.CompilerParams(
            dimension_semantics=("parallel", "arbitrary"),
            vmem_limit_bytes=55 * 1024 * 1024,
        ),
        name="mamba_layer_fused",
    )(x, g2, b2, inTz, wck, cb2, xpT, dtT, dtb2, An, Dp2, outT,
      msel, p0, dsel)
    return out


# speedup vs baseline: 1.0669x; 1.0669x over previous
import jax
import jax.numpy as jnp
from jax.experimental import pallas as pl
from jax.experimental.pallas import tpu as pltpu

_D_MODEL = 512
_D_INNER = 1024
_D_STATE = 16
_DT_RANK = 32
_D_CONV = 4
_T = 128


def _silu(v):
    return v * (1.0 / (1.0 + jnp.exp(-v)))


def _softplus(v):
    return jnp.maximum(v, 0.0) + jnp.log1p(jnp.exp(-jnp.abs(v)))


def _mamba_kernel(x_ref, g_ref, b_ref, inTz_ref, wck_ref, cb_ref, xpT_ref,
                  dtT_ref, dtb_ref, An_ref, Dp_ref, outT_ref, msel_ref,
                  p0_ref, dsel_ref, o_ref, carry_ref, h_ref, dA_ref, H_ref):
    lc = pl.program_id(1)

    @pl.when(lc == 0)
    def _():
        carry_ref[...] = jnp.zeros_like(carry_ref)
        h_ref[...] = jnp.zeros_like(h_ref)

    xb = x_ref[0]

    mu = jnp.mean(xb, axis=-1, keepdims=True)
    xc = xb - mu
    var = jnp.mean(xc * xc, axis=-1, keepdims=True)
    xn = xc * jax.lax.rsqrt(var + 1e-5) * g_ref[...] + b_ref[...]

    zg = jnp.dot(xn, inTz_ref[...], preferred_element_type=jnp.float32)

    x_ext = jnp.concatenate([carry_ref[5:8], xn],
                            axis=0).astype(jnp.bfloat16)
    carry_ref[5:8] = xn[_T - 3:, :]
    acc = jnp.broadcast_to(cb_ref[...], (_T, _D_INNER))
    for k in range(_D_CONV):
        acc = acc + jnp.dot(x_ext[k:k + _T], wck_ref[k],
                            preferred_element_type=jnp.float32)
    xcv = _silu(acc)

    x_dbl = jnp.dot(xcv, xpT_ref[...], preferred_element_type=jnp.float32)
    dtr = x_dbl[:, :_DT_RANK]
    Bc = x_dbl[:, _DT_RANK:_DT_RANK + _D_STATE]
    Cc = x_dbl[:, _DT_RANK + _D_STATE:2 * _D_STATE + _DT_RANK]

    dt_lin = jnp.dot(dtr, dtT_ref[...],
                     preferred_element_type=jnp.float32) + dtb_ref[...]
    dt_c = _softplus(dt_lin)
    dtx = dt_c * xcv

    _SC = 32
    An_v = An_ref[...]
    for t0 in range(0, _T, _SC):
        sl = slice(t0, t0 + _SC)
        dA_ref[sl, :_D_STATE] = jnp.exp(dt_c[sl, None, :] * An_v[None, :, :])
        dA_ref[sl, _D_STATE:] = Bc[sl, :, None] * dtx[sl, None, :]

    def sbody(t, h):
        ab = dA_ref[t]
        h2 = h * ab[:_D_STATE] + ab[_D_STATE:]
        H_ref[t] = h2
        return h2

    h_fin = jax.lax.fori_loop(0, _T, sbody, h_ref[...], unroll=16)
    h_ref[...] = h_fin

    cg8 = jnp.concatenate([Cc] * 8, axis=1)
    cgather = jnp.tile(cg8, (1, _D_STATE))
    mm = msel_ref[...] * cgather
    h_flat = H_ref[...].reshape(_T * _D_STATE, _D_INNER)
    ys = jnp.dot(mm, h_flat, preferred_element_type=jnp.float32)
    yy = ys + xcv * Dp_ref[...]
    yg = yy * _silu(zg)
    o_ref[0] = jnp.dot(yg, outT_ref[...], preferred_element_type=jnp.float32)


def kernel(x, ln_gamma, ln_beta, in_proj_w, conv_w, conv_b, x_proj_w,
           dt_proj_w, dt_proj_b, A_log, D_param, out_proj_w):
    B, L, D = x.shape
    n_chunks = L // _T

    inTz = in_proj_w[_D_INNER:, :].T
    Wx = in_proj_w[:_D_INNER, :].T
    wck = jnp.stack([Wx * conv_w[None, :, 0, k] for k in range(_D_CONV)],
                    axis=0).astype(jnp.bfloat16)
    xpT = x_proj_w.T
    dtT = dt_proj_w.T
    outT = out_proj_w.T
    An = -jnp.exp(A_log).T
    g2 = ln_gamma.reshape(1, D)
    b2 = ln_beta.reshape(1, D)
    cb2 = conv_b.reshape(1, _D_INNER)
    dtb2 = dt_proj_b.reshape(1, _D_INNER)
    Dp2 = D_param.reshape(1, _D_INNER)
    rows = jnp.arange(_T * _D_STATE)
    msel = (rows[None, :] // _D_STATE
            == jnp.arange(_T)[:, None]).astype(jnp.float32)
    p0 = (rows[:, None] // _D_STATE
          == jnp.arange(_T)[None, :]).astype(jnp.float32)
    dsel = (rows[:, None] % _D_STATE
            == jnp.arange(_D_STATE)[None, :]).astype(jnp.float32)

    full = lambda b, l: (0, 0)
    full3 = lambda b, l: (0, 0, 0)
    out = pl.pallas_call(
        _mamba_kernel,
        out_shape=jax.ShapeDtypeStruct((B, L, D), jnp.float32),
        grid=(B, n_chunks),
        in_specs=[
            pl.BlockSpec((1, _T, D), lambda b, l: (b, l, 0)),
            pl.BlockSpec((1, D), full),
            pl.BlockSpec((1, D), full),
            pl.BlockSpec((D, _D_INNER), full),
            pl.BlockSpec((_D_CONV, D, _D_INNER), full3),
            pl.BlockSpec((1, _D_INNER), full),
            pl.BlockSpec((_D_INNER, _DT_RANK + 2 * _D_STATE), full),
            pl.BlockSpec((_DT_RANK, _D_INNER), full),
            pl.BlockSpec((1, _D_INNER), full),
            pl.BlockSpec((_D_STATE, _D_INNER), full),
            pl.BlockSpec((1, _D_INNER), full),
            pl.BlockSpec((_D_INNER, D), full),
            pl.BlockSpec((_T, _T * _D_STATE), full),
            pl.BlockSpec((_T * _D_STATE, _T), full),
            pl.BlockSpec((_T * _D_STATE, _D_STATE), full),
        ],
        out_specs=pl.BlockSpec((1, _T, D), lambda b, l: (b, l, 0)),
        scratch_shapes=[
            pltpu.VMEM((8, _D_MODEL), jnp.float32),
            pltpu.VMEM((_D_STATE, _D_INNER), jnp.float32),
            pltpu.VMEM((_T, 2 * _D_STATE, _D_INNER), jnp.float32),
            pltpu.VMEM((_T, _D_STATE, _D_INNER), jnp.float32),
        ],
        compiler_params=pltpu.CompilerParams(
            dimension_semantics=("parallel", "arbitrary"),
            vmem_limit_bytes=55 * 1024 * 1024,
        ),
        name="mamba_layer_fused",
    )(x, g2, b2, inTz, wck, cb2, xpT, dtT, dtb2, An, Dp2, outT,
      msel, p0, dsel)
    return out


# speedup vs baseline: 1.1215x; 1.0512x over previous
import jax
import jax.numpy as jnp
from jax.experimental import pallas as pl
from jax.experimental.pallas import tpu as pltpu

_D_MODEL = 512
_D_INNER = 1024
_D_STATE = 16
_DT_RANK = 32
_D_CONV = 4
_T = 128


def _silu(v):
    return v * (1.0 / (1.0 + jnp.exp(-v)))


def _softplus(v):
    return jnp.maximum(v, 0.0) + jnp.log1p(jnp.exp(-jnp.abs(v)))


def _mamba_kernel(x_ref, g_ref, b_ref, inT_ref, cw_ref, cb_ref, xpT_ref,
                  dtT_ref, dtb_ref, An_ref, Dp_ref, outT_ref, msel_ref,
                  o_ref, carry_ref, h_ref, dA_ref, H_ref):
    lc = pl.program_id(1)

    @pl.when(lc == 0)
    def _():
        carry_ref[...] = jnp.zeros_like(carry_ref)
        h_ref[...] = jnp.zeros_like(h_ref)

    xb = x_ref[0]

    mu = jnp.mean(xb, axis=-1, keepdims=True)
    xc = xb - mu
    var = jnp.mean(xc * xc, axis=-1, keepdims=True)
    xn = xc * jax.lax.rsqrt(var + 1e-5) * g_ref[...] + b_ref[...]

    xz = jnp.dot(xn, inT_ref[...], preferred_element_type=jnp.float32)
    x_in = xz[:, :_D_INNER]
    zg = xz[:, _D_INNER:]

    x_ext = jnp.concatenate([carry_ref[5:8], x_in], axis=0)
    carry_ref[5:8] = x_in[_T - 3:, :]
    acc = jnp.broadcast_to(cb_ref[...], (_T, _D_INNER))
    for k in range(_D_CONV):
        acc = acc + x_ext[k:k + _T] * cw_ref[k:k + 1, :]
    xcv = _silu(acc)

    x_dbl = jnp.dot(xcv, xpT_ref[...], preferred_element_type=jnp.float32)
    dtr = x_dbl[:, :_DT_RANK]
    Bc = x_dbl[:, _DT_RANK:_DT_RANK + _D_STATE]
    Cc = x_dbl[:, _DT_RANK + _D_STATE:2 * _D_STATE + _DT_RANK]

    dt_lin = jnp.dot(dtr, dtT_ref[...],
                     preferred_element_type=jnp.float32) + dtb_ref[...]
    dt_c = _softplus(dt_lin)
    dtx = dt_c * xcv

    _SC = 32
    An_v = An_ref[...]
    for t0 in range(0, _T, _SC):
        sl = slice(t0, t0 + _SC)
        dA_ref[sl, :_D_STATE] = jnp.exp(dt_c[sl, None, :] * An_v[None, :, :])
        dA_ref[sl, _D_STATE:] = Bc[sl, :, None] * dtx[sl, None, :]

    def sbody(t, h):
        ab = dA_ref[t]
        h2 = h * ab[:_D_STATE] + ab[_D_STATE:]
        H_ref[t] = h2
        return h2

    h_fin = jax.lax.fori_loop(0, _T, sbody, h_ref[...], unroll=16)
    h_ref[...] = h_fin

    cg8 = jnp.concatenate([Cc] * 8, axis=1)
    cgather = jnp.tile(cg8, (1, _D_STATE))
    mm = msel_ref[...] * cgather
    h_flat = H_ref[...].reshape(_T * _D_STATE, _D_INNER)
    ys = jnp.dot(mm, h_flat, preferred_element_type=jnp.float32)
    yy = ys + xcv * Dp_ref[...]
    yg = yy * _silu(zg)
    o_ref[0] = jnp.dot(yg, outT_ref[...], preferred_element_type=jnp.float32)


def kernel(x, ln_gamma, ln_beta, in_proj_w, conv_w, conv_b, x_proj_w,
           dt_proj_w, dt_proj_b, A_log, D_param, out_proj_w):
    B, L, D = x.shape
    n_chunks = L // _T

    inT = in_proj_w.T
    xpT = x_proj_w.T
    dtT = dt_proj_w.T
    outT = out_proj_w.T
    An = -jnp.exp(A_log).T
    cw = jnp.transpose(conv_w[:, 0, :])
    g2 = ln_gamma.reshape(1, D)
    b2 = ln_beta.reshape(1, D)
    cb2 = conv_b.reshape(1, _D_INNER)
    dtb2 = dt_proj_b.reshape(1, _D_INNER)
    Dp2 = D_param.reshape(1, _D_INNER)
    msel = (jnp.arange(_T * _D_STATE)[None, :] // _D_STATE
            == jnp.arange(_T)[:, None]).astype(jnp.float32)

    full = lambda b, l: (0, 0)
    out = pl.pallas_call(
        _mamba_kernel,
        out_shape=jax.ShapeDtypeStruct((B, L, D), jnp.float32),
        grid=(B, n_chunks),
        in_specs=[
            pl.BlockSpec((1, _T, D), lambda b, l: (b, l, 0)),
            pl.BlockSpec((1, D), full),
            pl.BlockSpec((1, D), full),
            pl.BlockSpec((D, 2 * _D_INNER), full),
            pl.BlockSpec((_D_CONV, _D_INNER), full),
            pl.BlockSpec((1, _D_INNER), full),
            pl.BlockSpec((_D_INNER, _DT_RANK + 2 * _D_STATE), full),
            pl.BlockSpec((_DT_RANK, _D_INNER), full),
            pl.BlockSpec((1, _D_INNER), full),
            pl.BlockSpec((_D_STATE, _D_INNER), full),
            pl.BlockSpec((1, _D_INNER), full),
            pl.BlockSpec((_D_INNER, D), full),
            pl.BlockSpec((_T, _T * _D_STATE), full),
        ],
        out_specs=pl.BlockSpec((1, _T, D), lambda b, l: (b, l, 0)),
        scratch_shapes=[
            pltpu.VMEM((8, _D_INNER), jnp.float32),
            pltpu.VMEM((_D_STATE, _D_INNER), jnp.float32),
            pltpu.VMEM((_T, 2 * _D_STATE, _D_INNER), jnp.float32),
            pltpu.VMEM((_T, _D_STATE, _D_INNER), jnp.float32),
        ],
        compiler_params=pltpu.CompilerParams(
            dimension_semantics=("parallel", "arbitrary"),
            vmem_limit_bytes=50 * 1024 * 1024,
        ),
        name="mamba_layer_fused",
    )(x, g2, b2, inT, cw, cb2, xpT, dtT, dtb2, An, Dp2, outT, msel)
    return out


# speedup vs baseline: 1.3729x; 1.2241x over previous
import jax
import jax.numpy as jnp
from jax.experimental import pallas as pl
from jax.experimental.pallas import tpu as pltpu

_D_MODEL = 512
_D_INNER = 1024
_D_STATE = 16
_DT_RANK = 32
_D_CONV = 4
_T = 128


def _silu(v):
    return v * (1.0 / (1.0 + jnp.exp(-v)))


def _softplus(v):
    return jnp.maximum(v, 0.0) + jnp.log1p(jnp.exp(-jnp.abs(v)))


def _mamba_kernel(x_ref, g_ref, b_ref, inT_ref, cw_ref, cb_ref, xpT_ref,
                  dtT_ref, dtb_ref, An_ref, Dp_ref, outT_ref, msel_ref,
                  o_ref, carry_ref, h_ref, dA_ref, H_ref):
    lc = pl.program_id(1)

    @pl.when(lc == 0)
    def _():
        carry_ref[...] = jnp.zeros_like(carry_ref)
        h_ref[...] = jnp.zeros_like(h_ref)

    xb = x_ref[0]

    mu = jnp.mean(xb, axis=-1, keepdims=True)
    xc = xb - mu
    var = jnp.mean(xc * xc, axis=-1, keepdims=True)
    xn = xc * jax.lax.rsqrt(var + 1e-5) * g_ref[...] + b_ref[...]

    xz = jnp.dot(xn, inT_ref[...], preferred_element_type=jnp.float32)
    x_in = xz[:, :_D_INNER]
    zg = xz[:, _D_INNER:]

    x_ext = jnp.concatenate([carry_ref[5:8], x_in], axis=0)
    carry_ref[5:8] = x_in[_T - 3:, :]
    acc = jnp.broadcast_to(cb_ref[...], (_T, _D_INNER))
    for k in range(_D_CONV):
        acc = acc + x_ext[k:k + _T] * cw_ref[k:k + 1, :]
    xcv = _silu(acc)

    x_dbl = jnp.dot(xcv, xpT_ref[...], preferred_element_type=jnp.float32)
    dtr = x_dbl[:, :_DT_RANK]
    Bc = x_dbl[:, _DT_RANK:_DT_RANK + _D_STATE]
    Cc = x_dbl[:, _DT_RANK + _D_STATE:2 * _D_STATE + _DT_RANK]

    dt_lin = jnp.dot(dtr, dtT_ref[...],
                     preferred_element_type=jnp.float32) + dtb_ref[...]
    dt_c = _softplus(dt_lin)
    dtx = dt_c * xcv

    _SC = 32
    An_v = An_ref[...]
    for t0 in range(0, _T, _SC):
        sl = slice(t0, t0 + _SC)
        dA_ref[sl, :_D_STATE] = jnp.exp(dt_c[sl, None, :] * An_v[None, :, :])
        dA_ref[sl, _D_STATE:] = Bc[sl, :, None] * dtx[sl, None, :]

    h_run = h_ref[...]
    for t in range(_T):
        ab = dA_ref[t]
        h_run = h_run * ab[:_D_STATE] + ab[_D_STATE:]
        H_ref[t] = h_run
    h_ref[...] = h_run

    cg8 = jnp.concatenate([Cc] * 8, axis=1)
    cgather = jnp.tile(cg8, (1, _D_STATE))
    mm = msel_ref[...] * cgather
    h_flat = H_ref[...].reshape(_T * _D_STATE, _D_INNER)
    ys = jnp.dot(mm, h_flat, preferred_element_type=jnp.float32)
    yy = ys + xcv * Dp_ref[...]
    yg = yy * _silu(zg)
    o_ref[0] = jnp.dot(yg, outT_ref[...], preferred_element_type=jnp.float32)


def kernel(x, ln_gamma, ln_beta, in_proj_w, conv_w, conv_b, x_proj_w,
           dt_proj_w, dt_proj_b, A_log, D_param, out_proj_w):
    B, L, D = x.shape
    n_chunks = L // _T

    inT = in_proj_w.T
    xpT = x_proj_w.T
    dtT = dt_proj_w.T
    outT = out_proj_w.T
    An = -jnp.exp(A_log).T
    cw = jnp.transpose(conv_w[:, 0, :])
    g2 = ln_gamma.reshape(1, D)
    b2 = ln_beta.reshape(1, D)
    cb2 = conv_b.reshape(1, _D_INNER)
    dtb2 = dt_proj_b.reshape(1, _D_INNER)
    Dp2 = D_param.reshape(1, _D_INNER)
    msel = (jnp.arange(_T * _D_STATE)[None, :] // _D_STATE
            == jnp.arange(_T)[:, None]).astype(jnp.float32)

    full = lambda b, l: (0, 0)
    out = pl.pallas_call(
        _mamba_kernel,
        out_shape=jax.ShapeDtypeStruct((B, L, D), jnp.float32),
        grid=(B, n_chunks),
        in_specs=[
            pl.BlockSpec((1, _T, D), lambda b, l: (b, l, 0)),
            pl.BlockSpec((1, D), full),
            pl.BlockSpec((1, D), full),
            pl.BlockSpec((D, 2 * _D_INNER), full),
            pl.BlockSpec((_D_CONV, _D_INNER), full),
            pl.BlockSpec((1, _D_INNER), full),
            pl.BlockSpec((_D_INNER, _DT_RANK + 2 * _D_STATE), full),
            pl.BlockSpec((_DT_RANK, _D_INNER), full),
            pl.BlockSpec((1, _D_INNER), full),
            pl.BlockSpec((_D_STATE, _D_INNER), full),
            pl.BlockSpec((1, _D_INNER), full),
            pl.BlockSpec((_D_INNER, D), full),
            pl.BlockSpec((_T, _T * _D_STATE), full),
        ],
        out_specs=pl.BlockSpec((1, _T, D), lambda b, l: (b, l, 0)),
        scratch_shapes=[
            pltpu.VMEM((8, _D_INNER), jnp.float32),
            pltpu.VMEM((_D_STATE, _D_INNER), jnp.float32),
            pltpu.VMEM((_T, 2 * _D_STATE, _D_INNER), jnp.float32),
            pltpu.VMEM((_T, _D_STATE, _D_INNER), jnp.float32),
        ],
        compiler_params=pltpu.CompilerParams(
            dimension_semantics=("parallel", "arbitrary"),
            vmem_limit_bytes=50 * 1024 * 1024,
        ),
        name="mamba_layer_fused",
    )(x, g2, b2, inT, cw, cb2, xpT, dtT, dtb2, An, Dp2, outT, msel)
    return out


# speedup vs baseline: 1.4576x; 1.0617x over previous
import jax
import jax.numpy as jnp
from jax.experimental import pallas as pl
from jax.experimental.pallas import tpu as pltpu

_D_MODEL = 512
_D_INNER = 1024
_D_STATE = 16
_DT_RANK = 32
_D_CONV = 4
_T = 128


def _silu(v):
    return v * (1.0 / (1.0 + jnp.exp(-v)))


def _softplus(v):
    return jnp.maximum(v, 0.0) + jnp.log1p(jnp.exp(-jnp.abs(v)))


def _mamba_kernel(x_ref, g_ref, b_ref, inT_ref, cw_ref, cb_ref, xpT_ref,
                  dtT_ref, dtb_ref, An_ref, Dp_ref, outT_ref, msel_ref,
                  o_ref, carry_ref, h_ref, dA_ref, H_ref):
    lc = pl.program_id(1)

    @pl.when(lc == 0)
    def _():
        carry_ref[...] = jnp.zeros_like(carry_ref)
        h_ref[...] = jnp.zeros_like(h_ref)

    xb = x_ref[0]

    mu = jnp.mean(xb, axis=-1, keepdims=True)
    xc = xb - mu
    var = jnp.mean(xc * xc, axis=-1, keepdims=True)
    xn = xc * jax.lax.rsqrt(var + 1e-5) * g_ref[...] + b_ref[...]

    xz = jnp.dot(xn, inT_ref[...], preferred_element_type=jnp.float32)
    x_in = xz[:, :_D_INNER]
    zg = xz[:, _D_INNER:]

    x_ext = jnp.concatenate([carry_ref[5:8], x_in], axis=0)
    carry_ref[5:8] = x_in[_T - 3:, :]
    acc = jnp.broadcast_to(cb_ref[...], (_T, _D_INNER))
    for k in range(_D_CONV):
        acc = acc + x_ext[k:k + _T] * cw_ref[k:k + 1, :]
    xcv = _silu(acc)

    x_dbl = jnp.dot(xcv, xpT_ref[...], preferred_element_type=jnp.float32)
    dtr = x_dbl[:, :_DT_RANK]
    Bc = x_dbl[:, _DT_RANK:_DT_RANK + _D_STATE]
    Cc = x_dbl[:, _DT_RANK + _D_STATE:2 * _D_STATE + _DT_RANK]

    dt_lin = jnp.dot(dtr, dtT_ref[...],
                     preferred_element_type=jnp.float32) + dtb_ref[...]
    dt_c = _softplus(dt_lin)
    dtx = dt_c * xcv

    _SC = 32
    An_v = An_ref[...]
    for t0 in range(0, _T, _SC):
        sl = slice(t0, t0 + _SC)
        dA_ref[sl, :_D_STATE] = jnp.exp2(dt_c[sl, None, :] * An_v[None, :, :])
        dA_ref[sl, _D_STATE:] = Bc[sl, :, None] * dtx[sl, None, :]

    h_run = h_ref[...]
    for t in range(_T):
        ab = dA_ref[t]
        h_run = h_run * ab[:_D_STATE] + ab[_D_STATE:]
        H_ref[t] = h_run
    h_ref[...] = h_run

    cg8 = jnp.concatenate([Cc] * 8, axis=1)
    cgather = jnp.tile(cg8, (1, _D_STATE))
    mm = msel_ref[...] * cgather
    h_flat = H_ref[...].reshape(_T * _D_STATE, _D_INNER)
    ys = jnp.dot(mm, h_flat, preferred_element_type=jnp.float32)
    yy = ys + xcv * Dp_ref[...]
    yg = yy * _silu(zg)
    o_ref[0] = jnp.dot(yg, outT_ref[...], preferred_element_type=jnp.float32)


def kernel(x, ln_gamma, ln_beta, in_proj_w, conv_w, conv_b, x_proj_w,
           dt_proj_w, dt_proj_b, A_log, D_param, out_proj_w):
    B, L, D = x.shape
    n_chunks = L // _T

    inT = in_proj_w.T
    xpT = x_proj_w.T
    dtT = dt_proj_w.T
    outT = out_proj_w.T
    An = (-jnp.exp(A_log) * 1.4426950408889634).T
    cw = jnp.transpose(conv_w[:, 0, :])
    g2 = ln_gamma.reshape(1, D)
    b2 = ln_beta.reshape(1, D)
    cb2 = conv_b.reshape(1, _D_INNER)
    dtb2 = dt_proj_b.reshape(1, _D_INNER)
    Dp2 = D_param.reshape(1, _D_INNER)
    msel = (jnp.arange(_T * _D_STATE)[None, :] // _D_STATE
            == jnp.arange(_T)[:, None]).astype(jnp.float32)

    full = lambda b, l: (0, 0)
    out = pl.pallas_call(
        _mamba_kernel,
        out_shape=jax.ShapeDtypeStruct((B, L, D), jnp.float32),
        grid=(B, n_chunks),
        in_specs=[
            pl.BlockSpec((1, _T, D), lambda b, l: (b, l, 0)),
            pl.BlockSpec((1, D), full),
            pl.BlockSpec((1, D), full),
            pl.BlockSpec((D, 2 * _D_INNER), full),
            pl.BlockSpec((_D_CONV, _D_INNER), full),
            pl.BlockSpec((1, _D_INNER), full),
            pl.BlockSpec((_D_INNER, _DT_RANK + 2 * _D_STATE), full),
            pl.BlockSpec((_DT_RANK, _D_INNER), full),
            pl.BlockSpec((1, _D_INNER), full),
            pl.BlockSpec((_D_STATE, _D_INNER), full),
            pl.BlockSpec((1, _D_INNER), full),
            pl.BlockSpec((_D_INNER, D), full),
            pl.BlockSpec((_T, _T * _D_STATE), full),
        ],
        out_specs=pl.BlockSpec((1, _T, D), lambda b, l: (b, l, 0)),
        scratch_shapes=[
            pltpu.VMEM((8, _D_INNER), jnp.float32),
            pltpu.VMEM((_D_STATE, _D_INNER), jnp.float32),
            pltpu.VMEM((_T, 2 * _D_STATE, _D_INNER), jnp.float32),
            pltpu.VMEM((_T, _D_STATE, _D_INNER), jnp.float32),
        ],
        compiler_params=pltpu.CompilerParams(
            dimension_semantics=("parallel", "arbitrary"),
            vmem_limit_bytes=50 * 1024 * 1024,
        ),
        name="mamba_layer_fused",
    )(x, g2, b2, inT, cw, cb2, xpT, dtT, dtb2, An, Dp2, outT, msel)
    return out


# speedup vs baseline: 1.4951x; 1.0258x over previous
import jax
import jax.numpy as jnp
from jax.experimental import pallas as pl
from jax.experimental.pallas import tpu as pltpu

_D_MODEL = 512
_D_INNER = 1024
_D_STATE = 16
_DT_RANK = 32
_D_CONV = 4
_T = 128


def _silu(v):
    return v * (1.0 / (1.0 + jnp.exp(-v)))


def _softplus(v):
    return jnp.maximum(v, 0.0) + jnp.log1p(jnp.exp(-jnp.abs(v)))


def _mamba_kernel(x_ref, g_ref, b_ref, inT_ref, cw_ref, cb_ref, xpT_ref,
                  dtT_ref, dtb_ref, An_ref, Dp_ref, outT_ref, msel_ref,
                  o_ref, carry_ref, h_ref, dA_ref, H_ref):
    lc = pl.program_id(1)

    @pl.when(lc == 0)
    def _():
        carry_ref[...] = jnp.zeros_like(carry_ref)
        h_ref[...] = jnp.zeros_like(h_ref)

    xb = x_ref[0]

    mu = jnp.mean(xb, axis=-1, keepdims=True)
    xc = xb - mu
    var = jnp.mean(xc * xc, axis=-1, keepdims=True)
    xn = xc * jax.lax.rsqrt(var + 1e-5) * g_ref[...] + b_ref[...]

    xz = jnp.dot(xn.astype(jnp.bfloat16), inT_ref[...],
                 preferred_element_type=jnp.float32)
    x_in = xz[:, :_D_INNER]
    zg = xz[:, _D_INNER:]

    x_ext = jnp.concatenate([carry_ref[5:8], x_in], axis=0)
    carry_ref[5:8] = x_in[_T - 3:, :]
    acc = jnp.broadcast_to(cb_ref[...], (_T, _D_INNER))
    for k in range(_D_CONV):
        acc = acc + x_ext[k:k + _T] * cw_ref[k:k + 1, :]
    xcv = _silu(acc)

    x_dbl = jnp.dot(xcv, xpT_ref[...], preferred_element_type=jnp.float32)
    dtr = x_dbl[:, :_DT_RANK]
    Bc = x_dbl[:, _DT_RANK:_DT_RANK + _D_STATE]
    Cc = x_dbl[:, _DT_RANK + _D_STATE:2 * _D_STATE + _DT_RANK]

    dt_lin = jnp.dot(dtr, dtT_ref[...],
                     preferred_element_type=jnp.float32) + dtb_ref[...]
    dt_c = _softplus(dt_lin)
    dtx = dt_c * xcv

    _SC = 32
    An_v = An_ref[...]
    for t0 in range(0, _T, _SC):
        sl = slice(t0, t0 + _SC)
        dA_ref[sl, :_D_STATE] = jnp.exp2(dt_c[sl, None, :] * An_v[None, :, :])
        dA_ref[sl, _D_STATE:] = Bc[sl, :, None] * dtx[sl, None, :]

    h_run = h_ref[...]
    for t in range(_T):
        ab = dA_ref[t]
        h_run = h_run * ab[:_D_STATE] + ab[_D_STATE:]
        H_ref[t] = h_run
    h_ref[...] = h_run

    cg8 = jnp.concatenate([Cc] * 8, axis=1)
    cgather = jnp.tile(cg8, (1, _D_STATE))
    mm = msel_ref[...] * cgather
    h_flat = H_ref[...].reshape(_T * _D_STATE, _D_INNER)
    ys = jnp.dot(mm, h_flat, preferred_element_type=jnp.float32)
    yy = ys + xcv * Dp_ref[...]
    yg = yy * _silu(zg)
    o_ref[0] = jnp.dot(yg, outT_ref[...], preferred_element_type=jnp.float32)


def kernel(x, ln_gamma, ln_beta, in_proj_w, conv_w, conv_b, x_proj_w,
           dt_proj_w, dt_proj_b, A_log, D_param, out_proj_w):
    B, L, D = x.shape
    n_chunks = L // _T

    inT = in_proj_w.T.astype(jnp.bfloat16)
    xpT = x_proj_w.T
    dtT = dt_proj_w.T
    outT = out_proj_w.T
    An = (-jnp.exp(A_log) * 1.4426950408889634).T
    cw = jnp.transpose(conv_w[:, 0, :])
    g2 = ln_gamma.reshape(1, D)
    b2 = ln_beta.reshape(1, D)
    cb2 = conv_b.reshape(1, _D_INNER)
    dtb2 = dt_proj_b.reshape(1, _D_INNER)
    Dp2 = D_param.reshape(1, _D_INNER)
    msel = (jnp.arange(_T * _D_STATE)[None, :] // _D_STATE
            == jnp.arange(_T)[:, None]).astype(jnp.float32)

    full = lambda b, l: (0, 0)
    out = pl.pallas_call(
        _mamba_kernel,
        out_shape=jax.ShapeDtypeStruct((B, L, D), jnp.float32),
        grid=(B, n_chunks),
        in_specs=[
            pl.BlockSpec((1, _T, D), lambda b, l: (b, l, 0)),
            pl.BlockSpec((1, D), full),
            pl.BlockSpec((1, D), full),
            pl.BlockSpec((D, 2 * _D_INNER), full),
            pl.BlockSpec((_D_CONV, _D_INNER), full),
            pl.BlockSpec((1, _D_INNER), full),
            pl.BlockSpec((_D_INNER, _DT_RANK + 2 * _D_STATE), full),
            pl.BlockSpec((_DT_RANK, _D_INNER), full),
            pl.BlockSpec((1, _D_INNER), full),
            pl.BlockSpec((_D_STATE, _D_INNER), full),
            pl.BlockSpec((1, _D_INNER), full),
            pl.BlockSpec((_D_INNER, D), full),
            pl.BlockSpec((_T, _T * _D_STATE), full),
        ],
        out_specs=pl.BlockSpec((1, _T, D), lambda b, l: (b, l, 0)),
        scratch_shapes=[
            pltpu.VMEM((8, _D_INNER), jnp.float32),
            pltpu.VMEM((_D_STATE, _D_INNER), jnp.float32),
            pltpu.VMEM((_T, 2 * _D_STATE, _D_INNER), jnp.float32),
            pltpu.VMEM((_T, _D_STATE, _D_INNER), jnp.float32),
        ],
        compiler_params=pltpu.CompilerParams(
            dimension_semantics=("parallel", "arbitrary"),
            vmem_limit_bytes=50 * 1024 * 1024,
        ),
        name="mamba_layer_fused",
    )(x, g2, b2, inT, cw, cb2, xpT, dtT, dtb2, An, Dp2, outT, msel)
    return out
